# baseline (speedup 1.0000x reference)
import functools
import math

import jax
import jax.numpy as jnp
from jax import lax
from jax.experimental import pallas as pl
from jax.experimental.pallas import tpu as pltpu
from jax.experimental.pallas import tpu_sc as plsc

N = 10000
E = 320000
D_IN = 128
D_OUT = 64
NL = 100
WID = 80
CNT_COL = 64
R = 10240
NC = 2
NS = 16
NW = NC * NS
CH = 125
NSTREAM = 80
EPW = CH * NSTREAM
RPT = R // NS

B1 = 2000
B2 = 2000

S_SCALE = 64.0
MARG = 0.1
COS_M = math.cos(MARG)
SIN_M = math.sin(MARG)
TH = math.cos(math.pi - MARG)
MM = math.sin(math.pi - MARG) * MARG

_DN = (((1,), (1,)), ((), ()))


def _tc1_body(x_ref, wle_ref, yext_ref):
    xb = x_ref[...]
    y = lax.dot_general(xb, wle_ref[...], _DN,
                        preferred_element_type=jnp.float32,
                        precision=lax.Precision.HIGHEST)
    col = lax.broadcasted_iota(jnp.int32, (B1, WID), 1)
    ye = y + jnp.where(col == CNT_COL, 1.0, 0.0).astype(jnp.float32)
    yext_ref[...] = ye.astype(jnp.bfloat16)


def _tc1z_body(x_ref, wr_ref, z_ref):
    z_ref[...] = lax.dot_general(x_ref[...], wr_ref[...], _DN,
                                 preferred_element_type=jnp.float32,
                                 precision=lax.Precision.HIGHEST)


def _decode(w):
    ev = lax.bitcast_convert_type(lax.shift_left(w, 16), jnp.float32)
    od = lax.bitcast_convert_type(
        lax.bitwise_and(w, jnp.int32(-65536)), jnp.float32)
    return ev, od


def _tc2_body(s_ref, z_ref, bl_ref, lab_ref, w_ref, pe_ref, po_ref,
              feat_ref, out_ref):
    e0, o0 = _decode(s_ref[0])
    e1, o1 = _decode(s_ref[1])
    even = e0 + e1
    odd = o0 + o1
    lanes = lax.broadcasted_iota(jnp.int32, (B2, 128), 1)
    cnt = jnp.sum(jnp.where(lanes == CNT_COL // 2, even, 0.0),
                  axis=1, keepdims=True)
    cntc = jnp.maximum(cnt, 1.0)
    he = even[:, :D_OUT // 2]
    ho = odd[:, :D_OUT // 2]
    blp = bl_ref[...]
    zp = z_ref[...]
    h_e = he / cntc + blp[:, :D_OUT // 2] + zp[:, :D_OUT // 2]
    h_o = ho / cntc + blp[:, D_OUT // 2:] + zp[:, D_OUT // 2:]
    dn_f = (((1,), (0,)), ((), ()))
    feat_ref[...] = (
        lax.dot_general(h_e, pe_ref[...], dn_f,
                        preferred_element_type=jnp.float32,
                        precision=lax.Precision.HIGHEST)
        + lax.dot_general(h_o, po_ref[...], dn_f,
                          preferred_element_type=jnp.float32,
                          precision=lax.Precision.HIGHEST))
    hr_e = jnp.maximum(h_e, 0.0)
    hr_o = jnp.maximum(h_o, 0.0)
    n2 = (jnp.sum(hr_e * hr_e, axis=1, keepdims=True)
          + jnp.sum(hr_o * hr_o, axis=1, keepdims=True))
    inv = 1.0 / jnp.maximum(jnp.sqrt(n2), 1e-12)
    w = w_ref[...]
    wn = w / jnp.maximum(jnp.sqrt(jnp.sum(w * w, axis=1, keepdims=True)),
                         1e-12)
    cosine = (lax.dot_general(hr_e * inv, wn[:, :D_OUT // 2], _DN,
                              preferred_element_type=jnp.float32,
                              precision=lax.Precision.HIGHEST)
              + lax.dot_general(hr_o * inv, wn[:, D_OUT // 2:], _DN,
                                preferred_element_type=jnp.float32,
                                precision=lax.Precision.HIGHEST))
    sine = jnp.sqrt(jnp.clip(1.0 - cosine * cosine, 0.0, 1.0))
    phi = cosine * COS_M - sine * SIN_M
    phi = jnp.where(cosine > TH, phi, cosine - MM)
    lcols = lax.broadcasted_iota(jnp.int32, (B2, NL), 1)
    onehot = lcols == lab_ref[...]
    out_ref[...] = jnp.where(onehot, phi, cosine) * S_SCALE


def _sc_agg_body(yext_hbm, ei_hbm, zeros_hbm, out_hbm,
                 sidx_v, didx_v, rows_v, acc_s, gsem, ssem):
    cid = lax.axis_index("c")
    sid = lax.axis_index("s")
    wid = sid * NC + cid
    pltpu.sync_copy(zeros_hbm, acc_s.at[pl.ds(sid * RPT, RPT)])
    pltpu.sync_copy(ei_hbm.at[0, wid], sidx_v)
    pltpu.sync_copy(ei_hbm.at[1, wid], didx_v)
    plsc.subcore_barrier()

    def gather(j, k):
        pltpu.async_copy(yext_hbm.at[sidx_v.at[j]], rows_v[k], gsem[k])

    def gather_wait(j, k):
        pltpu.make_async_copy(yext_hbm.at[sidx_v.at[j]], rows_v[k],
                              gsem[k]).wait()

    def scatter(j, k):
        pltpu.async_copy(rows_v[k], acc_s.at[didx_v.at[j]], ssem[k],
                         add=True)

    def scatter_wait(j, k):
        pltpu.make_async_copy(rows_v[k], acc_s.at[didx_v.at[j]],
                              ssem[k]).wait()

    gather(0, 0)
    gather(1, 1)

    def body(i, carry):
        for k in range(4):
            j = 4 * i + k
            gather_wait(j, k)
            scatter(j, k)
            kk = (k + 2) % 4

            @pl.when(j + 2 < NSTREAM)
            def _():
                @pl.when(j >= 2)
                def _():
                    scatter_wait(j - 2, kk)

                gather(j + 2, kk)

        return carry

    lax.fori_loop(0, NSTREAM // 4, body, 0)
    for k in range(4):
        scatter_wait(NSTREAM - 4 + k, k)
    plsc.subcore_barrier()
    pltpu.sync_copy(acc_s.at[pl.ds(sid * RPT, RPT)],
                    out_hbm.at[cid, pl.ds(sid * RPT, RPT), pl.ds(0, WID)])


def _make_sc_agg():
    mesh = plsc.VectorSubcoreMesh(core_axis_name="c", subcore_axis_name="s")
    return pl.kernel(
        _sc_agg_body,
        out_type=jax.ShapeDtypeStruct((NC, R, 256), jnp.bfloat16),
        mesh=mesh,
        scratch_types=[
            pltpu.VMEM((NSTREAM, CH), jnp.int32),
            pltpu.VMEM((NSTREAM, CH), jnp.int32),
            [pltpu.VMEM((CH, WID), jnp.bfloat16) for _ in range(4)],
            pltpu.VMEM_SHARED((R, WID), jnp.bfloat16),
            [pltpu.SemaphoreType.DMA for _ in range(4)],
            [pltpu.SemaphoreType.DMA for _ in range(4)],
        ],
        compiler_params=pltpu.CompilerParams(use_tc_tiling_on_sc=False),
    )


_tc1_call = pl.pallas_call(
    _tc1_body,
    grid=(N // B1,),
    in_specs=[
        pl.BlockSpec((B1, D_IN), lambda j: (j, 0)),
        pl.BlockSpec((WID, D_IN), lambda j: (0, 0)),
    ],
    out_specs=pl.BlockSpec((B1, WID), lambda j: (j, 0)),
    out_shape=jax.ShapeDtypeStruct((R, WID), jnp.bfloat16),
)

_tc1z_call = pl.pallas_call(
    _tc1z_body,
    grid=(N // B1,),
    in_specs=[
        pl.BlockSpec((B1, D_IN), lambda j: (j, 0)),
        pl.BlockSpec((D_OUT, D_IN), lambda j: (0, 0)),
    ],
    out_specs=pl.BlockSpec((B1, D_OUT), lambda j: (j, 0)),
    out_shape=jax.ShapeDtypeStruct((N, D_OUT), jnp.float32),
)

_tc2_call = pl.pallas_call(
    _tc2_body,
    grid=(N // B2,),
    in_specs=[
        pl.BlockSpec((NC, B2, 128), lambda j: (0, j, 0)),
        pl.BlockSpec((B2, D_OUT), lambda j: (j, 0)),
        pl.BlockSpec((1, D_OUT), lambda j: (0, 0)),
        pl.BlockSpec((B2, 1), lambda j: (j, 0)),
        pl.BlockSpec((NL, D_OUT), lambda j: (0, 0)),
        pl.BlockSpec((D_OUT // 2, D_OUT), lambda j: (0, 0)),
        pl.BlockSpec((D_OUT // 2, D_OUT), lambda j: (0, 0)),
    ],
    out_specs=[
        pl.BlockSpec((B2, D_OUT), lambda j: (j, 0)),
        pl.BlockSpec((B2, NL), lambda j: (j, 0)),
    ],
    out_shape=[
        jax.ShapeDtypeStruct((N, D_OUT), jnp.float32),
        jax.ShapeDtypeStruct((N, NL), jnp.float32),
    ],
)


def kernel(x, edge_index, label, Wl, bl, Wr, weight):
    wle = jnp.zeros((WID, D_IN), jnp.float32).at[:D_OUT].set(Wl)
    ei4 = edge_index.reshape(2, NW, NSTREAM, CH)

    perm = jnp.concatenate([jnp.arange(0, D_OUT, 2), jnp.arange(1, D_OUT, 2)])
    eye = jnp.eye(D_OUT, dtype=jnp.float32)
    pe = eye[perm[:D_OUT // 2]]
    po = eye[perm[D_OUT // 2:]]

    yext = _tc1_call(x, wle)
    zp = _tc1z_call(x, Wr[perm])
    zeros = jnp.zeros((RPT, WID), jnp.bfloat16)
    part = _make_sc_agg()(yext, ei4, zeros)
    s_i32 = lax.bitcast_convert_type(part.reshape(NC, R, 128, 2), jnp.int32)
    feat, outp = _tc2_call(s_i32, zp, bl[perm].reshape(1, D_OUT),
                           label.reshape(N, 1), weight[:, perm], pe, po)
    return (feat, outp)


# speedup vs baseline: 1.7592x; 1.7592x over previous
import functools
import math

import jax
import jax.numpy as jnp
from jax import lax
from jax.experimental import pallas as pl
from jax.experimental.pallas import tpu as pltpu
from jax.experimental.pallas import tpu_sc as plsc

N = 10000
E = 320000
D_IN = 128
D_OUT = 64
NL = 100
WID = 80
CNT_COL = 64
R = 10240
NC = 2
NS = 16
NW = NC * NS
CH = 125
NSTREAM = 80
EPW = CH * NSTREAM
RPT = R // NS

B1 = 2000
B2 = 2000

S_SCALE = 64.0
MARG = 0.1
COS_M = math.cos(MARG)
SIN_M = math.sin(MARG)
TH = math.cos(math.pi - MARG)
MM = math.sin(math.pi - MARG) * MARG

_DN = (((1,), (1,)), ((), ()))


def _tc1_body(x_ref, wle_ref, yext_ref):
    xb = x_ref[...]
    y = lax.dot_general(xb, wle_ref[...], _DN,
                        preferred_element_type=jnp.float32,
                        precision=lax.Precision.HIGHEST)
    col = lax.broadcasted_iota(jnp.int32, (B1, WID), 1)
    ye = y + jnp.where(col == CNT_COL, 1.0, 0.0).astype(jnp.float32)
    yext_ref[...] = ye.astype(jnp.bfloat16)


def _tc1z_body(x_ref, wr_ref, z_ref):
    z_ref[...] = lax.dot_general(x_ref[...], wr_ref[...], _DN,
                                 preferred_element_type=jnp.float32,
                                 precision=lax.Precision.HIGHEST)


def _tc2_body(s_ref, z_ref, bl_ref, lab_ref, w_ref, feat_ref, out_ref):
    se = s_ref[0] + s_ref[1]
    cols = lax.broadcasted_iota(jnp.int32, (B2, WID), 1)
    cnt = jnp.sum(jnp.where(cols == CNT_COL, se, 0.0), axis=1, keepdims=True)
    agg = se[:, :D_OUT] / jnp.maximum(cnt, 1.0)
    h = agg + bl_ref[...] + z_ref[...]
    feat_ref[...] = h
    hr = jnp.maximum(h, 0.0)
    hn = hr / jnp.maximum(jnp.sqrt(jnp.sum(hr * hr, axis=1, keepdims=True)),
                          1e-12)
    w = w_ref[...]
    wn = w / jnp.maximum(jnp.sqrt(jnp.sum(w * w, axis=1, keepdims=True)),
                         1e-12)
    cosine = lax.dot_general(hn, wn, _DN,
                             preferred_element_type=jnp.float32,
                             precision=lax.Precision.HIGHEST)
    sine = jnp.sqrt(jnp.clip(1.0 - cosine * cosine, 0.0, 1.0))
    phi = cosine * COS_M - sine * SIN_M
    phi = jnp.where(cosine > TH, phi, cosine - MM)
    lcols = lax.broadcasted_iota(jnp.int32, (B2, NL), 1)
    onehot = lcols == lab_ref[...]
    out_ref[...] = jnp.where(onehot, phi, cosine) * S_SCALE


def _sc_agg_body(yext_hbm, ei_hbm, zeros_hbm, out_hbm,
                 sidx_v, didx_v, rows_v, acc_s, gsem, ssem):
    cid = lax.axis_index("c")
    sid = lax.axis_index("s")
    wid = sid * NC + cid
    pltpu.sync_copy(zeros_hbm, acc_s.at[pl.ds(sid * RPT, RPT)])
    pltpu.sync_copy(ei_hbm.at[0, wid], sidx_v)
    pltpu.sync_copy(ei_hbm.at[1, wid], didx_v)
    plsc.subcore_barrier()

    def gather(j, k):
        pltpu.async_copy(yext_hbm.at[sidx_v.at[j]], rows_v[k], gsem[k])

    def gather_wait(j, k):
        pltpu.make_async_copy(yext_hbm.at[sidx_v.at[j]], rows_v[k],
                              gsem[k]).wait()

    def scatter(j, k):
        pltpu.async_copy(rows_v[k], acc_s.at[didx_v.at[j]], ssem[k],
                         add=True)

    def scatter_wait(j, k):
        pltpu.make_async_copy(rows_v[k], acc_s.at[didx_v.at[j]],
                              ssem[k]).wait()

    gather(0, 0)
    gather(1, 1)

    def body(i, carry):
        for k in range(4):
            j = 4 * i + k
            gather_wait(j, k)
            scatter(j, k)
            kk = (k + 2) % 4

            @pl.when(j + 2 < NSTREAM)
            def _():
                @pl.when(j >= 2)
                def _():
                    scatter_wait(j - 2, kk)

                gather(j + 2, kk)

        return carry

    lax.fori_loop(0, NSTREAM // 4, body, 0)
    for k in range(4):
        scatter_wait(NSTREAM - 4 + k, k)
    plsc.subcore_barrier()
    pltpu.sync_copy(acc_s.at[pl.ds(sid * RPT, RPT)],
                    out_hbm.at[cid, pl.ds(sid * RPT, RPT)])


def _make_sc_agg():
    mesh = plsc.VectorSubcoreMesh(core_axis_name="c", subcore_axis_name="s")
    return pl.kernel(
        _sc_agg_body,
        out_type=jax.ShapeDtypeStruct((NC, R, WID), jnp.bfloat16),
        mesh=mesh,
        scratch_types=[
            pltpu.VMEM((NSTREAM, CH), jnp.int32),
            pltpu.VMEM((NSTREAM, CH), jnp.int32),
            [pltpu.VMEM((CH, WID), jnp.bfloat16) for _ in range(4)],
            pltpu.VMEM_SHARED((R, WID), jnp.bfloat16),
            [pltpu.SemaphoreType.DMA for _ in range(4)],
            [pltpu.SemaphoreType.DMA for _ in range(4)],
        ],
        compiler_params=pltpu.CompilerParams(use_tc_tiling_on_sc=False),
    )


_tc1_call = pl.pallas_call(
    _tc1_body,
    grid=(N // B1,),
    in_specs=[
        pl.BlockSpec((B1, D_IN), lambda j: (j, 0)),
        pl.BlockSpec((WID, D_IN), lambda j: (0, 0)),
    ],
    out_specs=pl.BlockSpec((B1, WID), lambda j: (j, 0)),
    out_shape=jax.ShapeDtypeStruct((R, WID), jnp.bfloat16),
)

_tc1z_call = pl.pallas_call(
    _tc1z_body,
    grid=(N // B1,),
    in_specs=[
        pl.BlockSpec((B1, D_IN), lambda j: (j, 0)),
        pl.BlockSpec((D_OUT, D_IN), lambda j: (0, 0)),
    ],
    out_specs=pl.BlockSpec((B1, D_OUT), lambda j: (j, 0)),
    out_shape=jax.ShapeDtypeStruct((N, D_OUT), jnp.float32),
)

_tc2_call = pl.pallas_call(
    _tc2_body,
    grid=(N // B2,),
    in_specs=[
        pl.BlockSpec((NC, B2, WID), lambda j: (0, j, 0)),
        pl.BlockSpec((B2, D_OUT), lambda j: (j, 0)),
        pl.BlockSpec((1, D_OUT), lambda j: (0, 0)),
        pl.BlockSpec((B2, 1), lambda j: (j, 0)),
        pl.BlockSpec((NL, D_OUT), lambda j: (0, 0)),
    ],
    out_specs=[
        pl.BlockSpec((B2, D_OUT), lambda j: (j, 0)),
        pl.BlockSpec((B2, NL), lambda j: (j, 0)),
    ],
    out_shape=[
        jax.ShapeDtypeStruct((N, D_OUT), jnp.float32),
        jax.ShapeDtypeStruct((N, NL), jnp.float32),
    ],
)


def kernel(x, edge_index, label, Wl, bl, Wr, weight):
    wle = jnp.zeros((WID, D_IN), jnp.float32).at[:D_OUT].set(Wl)
    ei4 = edge_index.reshape(2, NW, NSTREAM, CH)

    yext = _tc1_call(x, wle)
    z = _tc1z_call(x, Wr)
    zeros = jnp.zeros((RPT, WID), jnp.bfloat16)
    part = _make_sc_agg()(yext, ei4, zeros).astype(jnp.float32)
    feat, outp = _tc2_call(part, z, bl.reshape(1, D_OUT),
                           label.reshape(N, 1), weight)
    return (feat, outp)


# speedup vs baseline: 1.8026x; 1.0247x over previous
import functools
import math

import jax
import jax.numpy as jnp
from jax import lax
from jax.experimental import pallas as pl
from jax.experimental.pallas import tpu as pltpu
from jax.experimental.pallas import tpu_sc as plsc

N = 10000
E = 320000
D_IN = 128
D_OUT = 64
NL = 100
WID = 80
CNT_COL = 64
R = 10240
NC = 2
NS = 16
NW = NC * NS
CH = 125
NSTREAM = 80
EPW = CH * NSTREAM
RPT = R // NS

B1 = 2000
B2 = 2000

S_SCALE = 64.0
MARG = 0.1
COS_M = math.cos(MARG)
SIN_M = math.sin(MARG)
TH = math.cos(math.pi - MARG)
MM = math.sin(math.pi - MARG) * MARG

_DN = (((1,), (1,)), ((), ()))


def _tc1_body(x_ref, wle_ref, yext_ref):
    xb = x_ref[...]
    y = lax.dot_general(xb, wle_ref[...], _DN,
                        preferred_element_type=jnp.float32,
                        precision=lax.Precision.HIGHEST)
    col = lax.broadcasted_iota(jnp.int32, (B1, WID), 1)
    ye = y + jnp.where(col == CNT_COL, 1.0, 0.0).astype(jnp.float32)
    yext_ref[...] = ye.astype(jnp.bfloat16)


def _tc1z_body(x_ref, wr_ref, z_ref):
    z_ref[...] = lax.dot_general(x_ref[...], wr_ref[...], _DN,
                                 preferred_element_type=jnp.float32,
                                 precision=lax.Precision.HIGHEST)


def _tc2_body(s_ref, z_ref, bl_ref, lab_ref, w_ref, feat_ref, out_ref):
    se = s_ref[0].astype(jnp.float32) + s_ref[1].astype(jnp.float32)
    cols = lax.broadcasted_iota(jnp.int32, (B2, WID), 1)
    cnt = jnp.sum(jnp.where(cols == CNT_COL, se, 0.0), axis=1, keepdims=True)
    agg = se[:, :D_OUT] / jnp.maximum(cnt, 1.0)
    h = agg + bl_ref[...] + z_ref[...]
    feat_ref[...] = h
    hr = jnp.maximum(h, 0.0)
    hn = hr / jnp.maximum(jnp.sqrt(jnp.sum(hr * hr, axis=1, keepdims=True)),
                          1e-12)
    w = w_ref[...]
    wn = w / jnp.maximum(jnp.sqrt(jnp.sum(w * w, axis=1, keepdims=True)),
                         1e-12)
    cosine = lax.dot_general(hn, wn, _DN,
                             preferred_element_type=jnp.float32,
                             precision=lax.Precision.HIGHEST)
    sine = jnp.sqrt(jnp.clip(1.0 - cosine * cosine, 0.0, 1.0))
    phi = cosine * COS_M - sine * SIN_M
    phi = jnp.where(cosine > TH, phi, cosine - MM)
    lcols = lax.broadcasted_iota(jnp.int32, (B2, NL), 1)
    onehot = lcols == lab_ref[...]
    out_ref[...] = jnp.where(onehot, phi, cosine) * S_SCALE


def _sc_agg_body(yext_hbm, ei_hbm, zeros_hbm, out_hbm,
                 sidx_v, didx_v, rows_v, acc_s, gsem, ssem):
    cid = lax.axis_index("c")
    sid = lax.axis_index("s")
    wid = sid * NC + cid
    pltpu.sync_copy(zeros_hbm, acc_s.at[pl.ds(sid * RPT, RPT)])
    pltpu.sync_copy(ei_hbm.at[0, wid], sidx_v)
    pltpu.sync_copy(ei_hbm.at[1, wid], didx_v)
    plsc.subcore_barrier()

    def gather(j, k):
        pltpu.async_copy(yext_hbm.at[sidx_v.at[j]], rows_v[k], gsem[k])

    def gather_wait(j, k):
        pltpu.make_async_copy(yext_hbm.at[sidx_v.at[j]], rows_v[k],
                              gsem[k]).wait()

    def scatter(j, k):
        pltpu.async_copy(rows_v[k], acc_s.at[didx_v.at[j]], ssem[k],
                         add=True)

    def scatter_wait(j, k):
        pltpu.make_async_copy(rows_v[k], acc_s.at[didx_v.at[j]],
                              ssem[k]).wait()

    gather(0, 0)
    gather(1, 1)

    def body(i, carry):
        for k in range(4):
            j = 4 * i + k
            gather_wait(j, k)
            scatter(j, k)
            kk = (k + 2) % 4

            @pl.when(j + 2 < NSTREAM)
            def _():
                @pl.when(j >= 2)
                def _():
                    scatter_wait(j - 2, kk)

                gather(j + 2, kk)

        return carry

    lax.fori_loop(0, NSTREAM // 4, body, 0)
    for k in range(4):
        scatter_wait(NSTREAM - 4 + k, k)
    plsc.subcore_barrier()
    pltpu.sync_copy(acc_s.at[pl.ds(sid * RPT, RPT)],
                    out_hbm.at[cid, pl.ds(sid * RPT, RPT)])


def _make_sc_agg():
    mesh = plsc.VectorSubcoreMesh(core_axis_name="c", subcore_axis_name="s")
    return pl.kernel(
        _sc_agg_body,
        out_type=jax.ShapeDtypeStruct((NC, R, WID), jnp.bfloat16),
        mesh=mesh,
        scratch_types=[
            pltpu.VMEM((NSTREAM, CH), jnp.int32),
            pltpu.VMEM((NSTREAM, CH), jnp.int32),
            [pltpu.VMEM((CH, WID), jnp.bfloat16) for _ in range(4)],
            pltpu.VMEM_SHARED((R, WID), jnp.bfloat16),
            [pltpu.SemaphoreType.DMA for _ in range(4)],
            [pltpu.SemaphoreType.DMA for _ in range(4)],
        ],
        compiler_params=pltpu.CompilerParams(use_tc_tiling_on_sc=False),
    )


_tc1_call = pl.pallas_call(
    _tc1_body,
    grid=(N // B1,),
    in_specs=[
        pl.BlockSpec((B1, D_IN), lambda j: (j, 0)),
        pl.BlockSpec((WID, D_IN), lambda j: (0, 0)),
    ],
    out_specs=pl.BlockSpec((B1, WID), lambda j: (j, 0)),
    out_shape=jax.ShapeDtypeStruct((R, WID), jnp.bfloat16),
)

_tc1z_call = pl.pallas_call(
    _tc1z_body,
    grid=(N // B1,),
    in_specs=[
        pl.BlockSpec((B1, D_IN), lambda j: (j, 0)),
        pl.BlockSpec((D_OUT, D_IN), lambda j: (0, 0)),
    ],
    out_specs=pl.BlockSpec((B1, D_OUT), lambda j: (j, 0)),
    out_shape=jax.ShapeDtypeStruct((N, D_OUT), jnp.float32),
)

_tc2_call = pl.pallas_call(
    _tc2_body,
    grid=(N // B2,),
    in_specs=[
        pl.BlockSpec((NC, B2, WID), lambda j: (0, j, 0)),
        pl.BlockSpec((B2, D_OUT), lambda j: (j, 0)),
        pl.BlockSpec((1, D_OUT), lambda j: (0, 0)),
        pl.BlockSpec((B2, 1), lambda j: (j, 0)),
        pl.BlockSpec((NL, D_OUT), lambda j: (0, 0)),
    ],
    out_specs=[
        pl.BlockSpec((B2, D_OUT), lambda j: (j, 0)),
        pl.BlockSpec((B2, NL), lambda j: (j, 0)),
    ],
    out_shape=[
        jax.ShapeDtypeStruct((N, D_OUT), jnp.float32),
        jax.ShapeDtypeStruct((N, NL), jnp.float32),
    ],
)


def kernel(x, edge_index, label, Wl, bl, Wr, weight):
    wle = jnp.zeros((WID, D_IN), jnp.float32).at[:D_OUT].set(Wl)
    ei4 = edge_index.reshape(2, NW, NSTREAM, CH)

    yext = _tc1_call(x, wle)
    z = _tc1z_call(x, Wr)
    zeros = jnp.zeros((RPT, WID), jnp.bfloat16)
    part = _make_sc_agg()(yext, ei4, zeros)
    feat, outp = _tc2_call(part, z, bl.reshape(1, D_OUT),
                           label.reshape(N, 1), weight)
    return (feat, outp)


# speedup vs baseline: 1.8035x; 1.0005x over previous
import math

import jax
import jax.numpy as jnp
from jax import lax
from jax.experimental import pallas as pl
from jax.experimental.pallas import tpu as pltpu
from jax.experimental.pallas import tpu_sc as plsc

N = 10000
E = 320000
D_IN = 128
D_OUT = 64
NL = 100
WID = 80
CNT_COL = 64
R = 10240
NC = 2
NS = 16
NW = NC * NS
CH = 125
NSTREAM = 80
EPW = CH * NSTREAM
RPT = R // NS

B1 = 2000
B2 = 2000

S_SCALE = 64.0
MARG = 0.1
COS_M = math.cos(MARG)
SIN_M = math.sin(MARG)
TH = math.cos(math.pi - MARG)
MM = math.sin(math.pi - MARG) * MARG

_DN = (((1,), (1,)), ((), ()))


def _tc1_body(x_ref, wle_ref, yext_ref):
    xb = x_ref[...]
    y = lax.dot_general(xb, wle_ref[...], _DN,
                        preferred_element_type=jnp.float32,
                        precision=lax.Precision.HIGHEST)
    col = lax.broadcasted_iota(jnp.int32, (B1, WID), 1)
    ye = y + jnp.where(col == CNT_COL, 1.0, 0.0).astype(jnp.float32)
    yext_ref[...] = ye.astype(jnp.bfloat16)


def _tc1z_body(x_ref, wr_ref, z_ref):
    z_ref[...] = lax.dot_general(x_ref[...], wr_ref[...], _DN,
                                 preferred_element_type=jnp.float32,
                                 precision=lax.Precision.HIGHEST)


def _tc2_body(s_ref, z_ref, bl_ref, lab_ref, w_ref, feat_ref, out_ref):
    se = s_ref[0].astype(jnp.float32) + s_ref[1].astype(jnp.float32)
    cols = lax.broadcasted_iota(jnp.int32, (B2, WID), 1)
    cnt = jnp.sum(jnp.where(cols == CNT_COL, se, 0.0), axis=1, keepdims=True)
    agg = se[:, :D_OUT] / jnp.maximum(cnt, 1.0)
    h = agg + bl_ref[...] + z_ref[...]
    feat_ref[...] = h
    hr = jnp.maximum(h, 0.0)
    hn = hr / jnp.maximum(jnp.sqrt(jnp.sum(hr * hr, axis=1, keepdims=True)),
                          1e-12)
    w = w_ref[...]
    wn = w / jnp.maximum(jnp.sqrt(jnp.sum(w * w, axis=1, keepdims=True)),
                         1e-12)
    cosine = lax.dot_general(hn, wn, _DN,
                             preferred_element_type=jnp.float32,
                             precision=lax.Precision.HIGHEST)
    sine = jnp.sqrt(jnp.clip(1.0 - cosine * cosine, 0.0, 1.0))
    phi = cosine * COS_M - sine * SIN_M
    phi = jnp.where(cosine > TH, phi, cosine - MM)
    lcols = lax.broadcasted_iota(jnp.int32, (B2, NL), 1)
    onehot = lcols == lab_ref[...]
    out_ref[...] = jnp.where(onehot, phi, cosine) * S_SCALE


def _sc_agg_body(yext_hbm, ei_hbm, zeros_hbm, out_hbm,
                 sidx_v, didx_v, rows_v, acc_s, gsem, ssem):
    cid = lax.axis_index("c")
    sid = lax.axis_index("s")
    wid = sid * NC + cid
    pltpu.sync_copy(zeros_hbm, acc_s.at[pl.ds(sid * RPT, RPT)])
    pltpu.sync_copy(ei_hbm.at[0, wid], sidx_v)
    pltpu.sync_copy(ei_hbm.at[1, wid], didx_v)
    plsc.subcore_barrier()

    def gather(j, k):
        pltpu.async_copy(yext_hbm.at[sidx_v.at[j]], rows_v[k], gsem[k])

    def gather_wait(j, k):
        pltpu.make_async_copy(yext_hbm.at[sidx_v.at[j]], rows_v[k],
                              gsem[k]).wait()

    def scatter(j, k):
        pltpu.async_copy(rows_v[k], acc_s.at[didx_v.at[j]], ssem[k],
                         add=True)

    def scatter_wait(j, k):
        pltpu.make_async_copy(rows_v[k], acc_s.at[didx_v.at[j]],
                              ssem[k]).wait()

    gather(0, 0)
    gather(1, 1)

    def body(i, carry):
        for k in range(4):
            j = 4 * i + k
            gather_wait(j, k)
            scatter(j, k)
            kk = (k + 2) % 4

            @pl.when(j + 2 < NSTREAM)
            def _():
                @pl.when(j >= 2)
                def _():
                    scatter_wait(j - 2, kk)

                gather(j + 2, kk)

        return carry

    lax.fori_loop(0, NSTREAM // 4, body, 0)
    for k in range(4):
        scatter_wait(NSTREAM - 4 + k, k)
    plsc.subcore_barrier()
    pltpu.sync_copy(acc_s.at[pl.ds(sid * RPT, RPT)],
                    out_hbm.at[cid, pl.ds(sid * RPT, RPT)])


def _make_sc_agg():
    mesh = plsc.VectorSubcoreMesh(core_axis_name="c", subcore_axis_name="s")
    return pl.kernel(
        _sc_agg_body,
        out_type=jax.ShapeDtypeStruct((NC, R, WID), jnp.bfloat16),
        mesh=mesh,
        scratch_types=[
            pltpu.VMEM((NSTREAM, CH), jnp.int32),
            pltpu.VMEM((NSTREAM, CH), jnp.int32),
            [pltpu.VMEM((CH, WID), jnp.bfloat16) for _ in range(4)],
            pltpu.VMEM_SHARED((R, WID), jnp.bfloat16),
            [pltpu.SemaphoreType.DMA for _ in range(4)],
            [pltpu.SemaphoreType.DMA for _ in range(4)],
        ],
        compiler_params=pltpu.CompilerParams(use_tc_tiling_on_sc=False),
    )


_tc1_call = pl.pallas_call(
    _tc1_body,
    grid=(N // B1,),
    in_specs=[
        pl.BlockSpec((B1, D_IN), lambda j: (j, 0)),
        pl.BlockSpec((WID, D_IN), lambda j: (0, 0)),
    ],
    out_specs=pl.BlockSpec((B1, WID), lambda j: (j, 0)),
    out_shape=jax.ShapeDtypeStruct((R, WID), jnp.bfloat16),
)

_tc1z_call = pl.pallas_call(
    _tc1z_body,
    grid=(N // B1,),
    in_specs=[
        pl.BlockSpec((B1, D_IN), lambda j: (j, 0)),
        pl.BlockSpec((D_OUT, D_IN), lambda j: (0, 0)),
    ],
    out_specs=pl.BlockSpec((B1, D_OUT), lambda j: (j, 0)),
    out_shape=jax.ShapeDtypeStruct((N, D_OUT), jnp.float32),
)

_tc2_call = pl.pallas_call(
    _tc2_body,
    grid=(N // B2,),
    in_specs=[
        pl.BlockSpec((NC, B2, WID), lambda j: (0, j, 0)),
        pl.BlockSpec((B2, D_OUT), lambda j: (j, 0)),
        pl.BlockSpec((1, D_OUT), lambda j: (0, 0)),
        pl.BlockSpec((B2, 1), lambda j: (j, 0)),
        pl.BlockSpec((NL, D_OUT), lambda j: (0, 0)),
    ],
    out_specs=[
        pl.BlockSpec((B2, D_OUT), lambda j: (j, 0)),
        pl.BlockSpec((B2, NL), lambda j: (j, 0)),
    ],
    out_shape=[
        jax.ShapeDtypeStruct((N, D_OUT), jnp.float32),
        jax.ShapeDtypeStruct((N, NL), jnp.float32),
    ],
)


def kernel(x, edge_index, label, Wl, bl, Wr, weight):
    wle = jnp.zeros((WID, D_IN), jnp.float32).at[:D_OUT].set(Wl)
    ei4 = edge_index.reshape(2, NW, NSTREAM, CH)

    yext = _tc1_call(x, wle)
    z = _tc1z_call(x, Wr)
    zeros = jnp.zeros((RPT, WID), jnp.bfloat16)
    part = _make_sc_agg()(yext, ei4, zeros)
    feat, outp = _tc2_call(part, z, bl.reshape(1, D_OUT),
                           label.reshape(N, 1), weight)
    return (feat, outp)
